# baseline (speedup 1.0000x reference)
import functools

import jax
import jax.numpy as jnp
from jax import lax
from jax.experimental import pallas as pl
from jax.experimental.pallas import tpu as pltpu
from jax.experimental.pallas import tpu_sc as plsc

N = 10000
E = 160000
D = 256
DH = 128

NC = 2
NS = 16
EPT = E // NS
BLK = 80
NBLK = EPT // BLK
NBUF = 2
ROWS_A = 632
ROWS_B = N - 15 * ROWS_A

_mesh = plsc.VectorSubcoreMesh(core_axis_name="c", subcore_axis_name="s")


def _deg_body(row2_hbm, col2_hbm, rdeg_hbm, cdeg_hbm,
              hist_sh, idx_v, ones_v, zbuf_v, sem):
    c = lax.axis_index("c")
    s = lax.axis_index("s")

    for i in range((BLK + 15) // 16):
        ones_v[pl.ds(i * 16, 16)] = jnp.ones((16,), jnp.float32)
    for i in range(640 // 16):
        zbuf_v[pl.ds(i * 16, 16)] = jnp.zeros((16,), jnp.float32)

    @pl.when(s < NS - 1)
    def _():
        pltpu.sync_copy(zbuf_v.at[pl.ds(0, 640)], hist_sh.at[pl.ds(s * 640, 640)])

    @pl.when(s == NS - 1)
    def _():
        pltpu.sync_copy(zbuf_v.at[pl.ds(0, 400)], hist_sh.at[pl.ds(9600, 400)])

    plsc.subcore_barrier()

    def _hist(src3d, out_hbm):
        done = 0
        while done < NBLK:
            nb = min(40, NBLK - done)
            pltpu.sync_copy(src3d.at[s].at[pl.ds(done, nb)],
                            idx_v.at[pl.ds(0, nb)])

            def _fire(i, carry):
                pltpu.async_copy(ones_v.at[pl.ds(0, BLK)],
                                 hist_sh.at[idx_v.at[i]], sem, add=True)
                return carry

            def _drain(i, carry):
                pltpu.make_async_copy(ones_v.at[pl.ds(0, BLK)],
                                      hist_sh.at[idx_v.at[i]], sem).wait()
                return carry

            lax.fori_loop(0, nb, _fire, 0)
            lax.fori_loop(0, nb, _drain, 0)
            done += nb
        plsc.subcore_barrier()

        @pl.when(s < NS - 1)
        def _():
            pltpu.sync_copy(hist_sh.at[pl.ds(s * 640, 640)],
                            zbuf_v.at[pl.ds(0, 640)])
            pltpu.sync_copy(zbuf_v.at[pl.ds(0, 640)],
                            out_hbm.at[pl.ds(s * 640, 640)])

        @pl.when(s == NS - 1)
        def _():
            pltpu.sync_copy(hist_sh.at[pl.ds(9600, 400)],
                            zbuf_v.at[pl.ds(0, 400)])
            pltpu.sync_copy(zbuf_v.at[pl.ds(0, 400)],
                            out_hbm.at[pl.ds(9600, 400)])

    @pl.when(c == 0)
    def _():
        _hist(row2_hbm, rdeg_hbm)

    @pl.when(c == 1)
    def _():
        _hist(col2_hbm, cdeg_hbm)


_deg_call = functools.partial(
    pl.kernel,
    out_type=(jax.ShapeDtypeStruct((N,), jnp.float32),
              jax.ShapeDtypeStruct((N,), jnp.float32)),
    mesh=_mesh,
    scratch_types=(
        pltpu.VMEM_SHARED((N,), jnp.float32),
        pltpu.VMEM((40, BLK), jnp.int32),
        pltpu.VMEM(((BLK + 15) // 16 * 16,), jnp.float32),
        pltpu.VMEM((640,), jnp.float32),
        pltpu.SemaphoreType.DMA,
    ),
)(_deg_body)


def _norm_body(feat_ref, rdeg_ref, z0_ref, z1_ref):
    nl = lax.rsqrt(jnp.maximum(rdeg_ref[...], 1.0))
    z0_ref[...] = feat_ref[:, :DH] * nl
    z1_ref[...] = feat_ref[:, DH:] * nl


def _norm_call(feat, rdeg2d):
    br = 1000
    return pl.pallas_call(
        _norm_body,
        grid=(N // br,),
        in_specs=[pl.BlockSpec((br, D), lambda i: (i, 0)),
                  pl.BlockSpec((br, 1), lambda i: (i, 0))],
        out_specs=[pl.BlockSpec((br, DH), lambda i: (i, 0)),
                   pl.BlockSpec((br, DH), lambda i: (i, 0))],
        out_shape=[jax.ShapeDtypeStruct((N, DH), jnp.float32),
                   jax.ShapeDtypeStruct((N, DH), jnp.float32)],
    )(feat, rdeg2d)


NBLKP = 128


def _spmm_body(z0_hbm, z1_hbm, col_hbm, row4_hbm, z2d_hbm, a0_hbm, a1_hbm,
               agg_sh, colidx_v, rowidx_v, gbuf_v,
               gsem0, gsem1, ssem0, ssem1):
    c = lax.axis_index("c")
    s = lax.axis_index("s")
    gsems = (gsem0, gsem1)
    ssems = (ssem0, ssem1)

    CH_A = (80,) * 7 + (72,)
    CH_B = (80,) * 6 + (40,)

    def _zero_rows(base, sizes):
        off = 0
        for sz in sizes:
            pltpu.sync_copy(gbuf_v.at[0].at[pl.ds(0, sz)],
                            agg_sh.at[pl.ds(base + off, sz)])
            off += sz

    def _drain_rows(base, sizes, out_hbm):
        off = 0
        for sz in sizes:
            pltpu.sync_copy(agg_sh.at[pl.ds(base + off, sz)],
                            gbuf_v.at[0].at[pl.ds(0, sz)])
            pltpu.sync_copy(gbuf_v.at[0].at[pl.ds(0, sz)],
                            out_hbm.at[pl.ds(base + off, sz)])
            off += sz

    def _flow(zt_hbm, out_hbm):
        pltpu.sync_copy(z2d_hbm, gbuf_v.at[0])

        @pl.when(s < NS - 1)
        def _():
            _zero_rows(s * ROWS_A, CH_A)

        @pl.when(s == NS - 1)
        def _():
            _zero_rows(15 * ROWS_A, CH_B)

        plsc.subcore_barrier()

        def _fire_g(j, b):
            pltpu.async_copy(
                zt_hbm.at[colidx_v.at[pl.ds(j * BLK, BLK)]],
                gbuf_v.at[b], gsems[b])

        def _wait_g(j, b):
            pltpu.make_async_copy(
                zt_hbm.at[colidx_v.at[pl.ds(j * BLK, BLK)]],
                gbuf_v.at[b], gsems[b]).wait()

        def _fire_s(j, b):
            pltpu.async_copy(gbuf_v.at[b], agg_sh.at[rowidx_v.at[j]],
                             ssems[b], add=True)

        def _wait_s(j, b):
            pltpu.make_async_copy(gbuf_v.at[b],
                                  agg_sh.at[rowidx_v.at[j]],
                                  ssems[b]).wait()

        pltpu.sync_copy(col_hbm.at[pl.ds(s * EPT, EPT)], colidx_v)
        pltpu.sync_copy(row4_hbm.at[s], rowidx_v)
        _fire_g(0, 0)
        _fire_g(1, 1)

        def _scat_sync(j, b):
            pltpu.sync_copy(gbuf_v.at[b], agg_sh.at[rowidx_v.at[j]],
                            add=True)

        def _outer(g, carry):
            j0 = g * NBUF
            _wait_g(j0, 0)
            _scat_sync(j0, 0)
            _fire_g(j0 + NBUF, 0)
            j1 = j0 + 1
            _wait_g(j1, 1)
            _scat_sync(j1, 1)

            @pl.when(g < (NBLK - 1) // NBUF - 1)
            def _():
                _fire_g(j1 + NBUF, 1)

            return carry

        lax.fori_loop(0, (NBLK - 1) // NBUF, _outer, 0)
        _wait_g(NBLK - 1, 0)
        _scat_sync(NBLK - 1, 0)
        plsc.subcore_barrier()

        @pl.when(s < NS - 1)
        def _():
            _drain_rows(s * ROWS_A, CH_A, out_hbm)

        @pl.when(s == NS - 1)
        def _():
            _drain_rows(15 * ROWS_A, CH_B, out_hbm)

    @pl.when(c == 0)
    def _():
        _flow(z0_hbm, a0_hbm)

    @pl.when(c == 1)
    def _():
        _flow(z1_hbm, a1_hbm)


_spmm_call = functools.partial(
    pl.kernel,
    out_type=(jax.ShapeDtypeStruct((N, DH), jnp.float32),
              jax.ShapeDtypeStruct((N, DH), jnp.float32)),
    mesh=_mesh,
    scratch_types=(
        pltpu.VMEM_SHARED((N, DH), jnp.float32),
        pltpu.VMEM((EPT,), jnp.int32),
        pltpu.VMEM((NBLKP, BLK), jnp.int32),
        pltpu.VMEM((NBUF, BLK, DH), jnp.float32),
        pltpu.SemaphoreType.DMA,
        pltpu.SemaphoreType.DMA,
        pltpu.SemaphoreType.DMA,
        pltpu.SemaphoreType.DMA,
    ),
)(_spmm_body)


def _out_body(a0_ref, a1_ref, w_ref, cdeg_ref, b_ref, o_ref):
    a = jnp.concatenate([a0_ref[...], a1_ref[...]], axis=1)
    y = lax.dot_general(a, w_ref[...], (((1,), (0,)), ((), ())),
                        preferred_element_type=jnp.float32,
                        precision=lax.Precision.HIGHEST)
    nr = lax.rsqrt(jnp.maximum(cdeg_ref[...], 1.0))
    o_ref[...] = y * nr + b_ref[...]


def _out_call(a0, a1, W, cdeg2d, bias2d):
    br = 1000
    return pl.pallas_call(
        _out_body,
        grid=(N // br,),
        in_specs=[pl.BlockSpec((br, DH), lambda i: (i, 0)),
                  pl.BlockSpec((br, DH), lambda i: (i, 0)),
                  pl.BlockSpec((D, D), lambda i: (0, 0)),
                  pl.BlockSpec((br, 1), lambda i: (i, 0)),
                  pl.BlockSpec((1, D), lambda i: (0, 0))],
        out_specs=pl.BlockSpec((br, D), lambda i: (i, 0)),
        out_shape=jax.ShapeDtypeStruct((N, D), jnp.float32),
    )(a0, a1, W, cdeg2d, bias2d)


def kernel(feat, edge_index, W, bias):
    row = edge_index[0].astype(jnp.int32)
    col = edge_index[1].astype(jnp.int32)
    row3 = row.reshape(NS, NBLK, BLK)
    col3 = col.reshape(NS, NBLK, BLK)
    row4 = jnp.pad(row3, ((0, 0), (0, NBLKP - NBLK), (0, 0)))
    z2d = jnp.zeros((BLK, DH), jnp.float32)

    rdeg, cdeg = _deg_call(row3, col3)
    z0, z1 = _norm_call(feat, rdeg.reshape(N, 1))
    a0, a1 = _spmm_call(z0, z1, col, row4, z2d)
    return _out_call(a0, a1, W, cdeg.reshape(N, 1), bias.reshape(1, D))


# speedup vs baseline: 1.0234x; 1.0234x over previous
import functools

import jax
import jax.numpy as jnp
from jax import lax
from jax.experimental import pallas as pl
from jax.experimental.pallas import tpu as pltpu
from jax.experimental.pallas import tpu_sc as plsc

N = 10000
E = 160000
D = 256
DH = 128

NC = 2
NS = 16
EPT = E // NS
BLK = 80
NBLK = EPT // BLK
NBUF = 2
ROWS_A = 640
ROWS_B = N - 15 * ROWS_A

_mesh = plsc.VectorSubcoreMesh(core_axis_name="c", subcore_axis_name="s")


def _deg_body(row2_hbm, col2_hbm, rdeg_hbm, cdeg_hbm,
              hist_sh, idx_v, ones_v, zbuf_v, sem):
    c = lax.axis_index("c")
    s = lax.axis_index("s")

    for i in range((BLK + 15) // 16):
        ones_v[pl.ds(i * 16, 16)] = jnp.ones((16,), jnp.float32)
    for i in range(640 // 16):
        zbuf_v[pl.ds(i * 16, 16)] = jnp.zeros((16,), jnp.float32)

    @pl.when(s < NS - 1)
    def _():
        pltpu.sync_copy(zbuf_v.at[pl.ds(0, 640)], hist_sh.at[pl.ds(s * 640, 640)])

    @pl.when(s == NS - 1)
    def _():
        pltpu.sync_copy(zbuf_v.at[pl.ds(0, 400)], hist_sh.at[pl.ds(9600, 400)])

    plsc.subcore_barrier()

    def _hist(src3d, out_hbm):
        done = 0
        while done < NBLK:
            nb = min(40, NBLK - done)
            pltpu.sync_copy(src3d.at[s].at[pl.ds(done, nb)],
                            idx_v.at[pl.ds(0, nb)])

            def _fire(i, carry):
                pltpu.async_copy(ones_v.at[pl.ds(0, BLK)],
                                 hist_sh.at[idx_v.at[i]], sem, add=True)
                return carry

            def _drain(i, carry):
                pltpu.make_async_copy(ones_v.at[pl.ds(0, BLK)],
                                      hist_sh.at[idx_v.at[i]], sem).wait()
                return carry

            lax.fori_loop(0, nb, _fire, 0)
            lax.fori_loop(0, nb, _drain, 0)
            done += nb
        plsc.subcore_barrier()

        @pl.when(s < NS - 1)
        def _():
            pltpu.sync_copy(hist_sh.at[pl.ds(s * 640, 640)],
                            zbuf_v.at[pl.ds(0, 640)])
            pltpu.sync_copy(zbuf_v.at[pl.ds(0, 640)],
                            out_hbm.at[pl.ds(s * 640, 640)])

        @pl.when(s == NS - 1)
        def _():
            pltpu.sync_copy(hist_sh.at[pl.ds(9600, 400)],
                            zbuf_v.at[pl.ds(0, 400)])
            pltpu.sync_copy(zbuf_v.at[pl.ds(0, 400)],
                            out_hbm.at[pl.ds(9600, 400)])

    @pl.when(c == 0)
    def _():
        _hist(row2_hbm, rdeg_hbm)

    @pl.when(c == 1)
    def _():
        _hist(col2_hbm, cdeg_hbm)


_deg_call = functools.partial(
    pl.kernel,
    out_type=(jax.ShapeDtypeStruct((N,), jnp.float32),
              jax.ShapeDtypeStruct((N,), jnp.float32)),
    mesh=_mesh,
    scratch_types=(
        pltpu.VMEM_SHARED((N,), jnp.float32),
        pltpu.VMEM((40, BLK), jnp.int32),
        pltpu.VMEM(((BLK + 15) // 16 * 16,), jnp.float32),
        pltpu.VMEM((640,), jnp.float32),
        pltpu.SemaphoreType.DMA,
    ),
)(_deg_body)


def _norm_body(feat_ref, rdeg_ref, z0_ref, z1_ref):
    nl = lax.rsqrt(jnp.maximum(rdeg_ref[...], 1.0))
    z0_ref[...] = feat_ref[:, :DH] * nl
    z1_ref[...] = feat_ref[:, DH:] * nl


def _norm_call(feat, rdeg2d):
    br = 2000
    return pl.pallas_call(
        _norm_body,
        grid=(N // br,),
        in_specs=[pl.BlockSpec((br, D), lambda i: (i, 0)),
                  pl.BlockSpec((br, 1), lambda i: (i, 0))],
        out_specs=[pl.BlockSpec((br, DH), lambda i: (i, 0)),
                   pl.BlockSpec((br, DH), lambda i: (i, 0))],
        out_shape=[jax.ShapeDtypeStruct((N, DH), jnp.float32),
                   jax.ShapeDtypeStruct((N, DH), jnp.float32)],
    )(feat, rdeg2d)


def _spmm_body(z0_hbm, z1_hbm, col_hbm, row4_hbm, z2d_hbm, a0_hbm, a1_hbm,
               agg_sh, colidx_v, rowidx_v, gbuf_v,
               gsem0, gsem1, ssem0, ssem1):
    c = lax.axis_index("c")
    s = lax.axis_index("s")
    gsems = (gsem0, gsem1)
    ssems = (ssem0, ssem1)

    CH_A = (80,) * 8
    CH_B = (80,) * 5

    def _zero_rows(base, sizes):
        off = 0
        for sz in sizes:
            pltpu.sync_copy(gbuf_v.at[0].at[pl.ds(0, sz)],
                            agg_sh.at[pl.ds(base + off, sz)])
            off += sz

    def _drain_rows(base, sizes, out_hbm):
        off = 0
        for sz in sizes:
            pltpu.sync_copy(agg_sh.at[pl.ds(base + off, sz)],
                            gbuf_v.at[0].at[pl.ds(0, sz)])
            pltpu.sync_copy(gbuf_v.at[0].at[pl.ds(0, sz)],
                            out_hbm.at[pl.ds(base + off, sz)])
            off += sz

    def _flow(zt_hbm, out_hbm):
        pltpu.sync_copy(z2d_hbm, gbuf_v.at[0])

        @pl.when(s < NS - 1)
        def _():
            _zero_rows(s * ROWS_A, CH_A)

        @pl.when(s == NS - 1)
        def _():
            _zero_rows(15 * ROWS_A, CH_B)

        plsc.subcore_barrier()

        def _fire_g(j, b):
            pltpu.async_copy(
                zt_hbm.at[colidx_v.at[pl.ds(j * BLK, BLK)]],
                gbuf_v.at[b], gsems[b])

        def _wait_g(j, b):
            pltpu.make_async_copy(
                zt_hbm.at[colidx_v.at[pl.ds(j * BLK, BLK)]],
                gbuf_v.at[b], gsems[b]).wait()

        def _fire_s(j, b):
            pltpu.async_copy(gbuf_v.at[b], agg_sh.at[rowidx_v.at[j]],
                             ssems[b], add=True)

        def _wait_s(j, b):
            pltpu.make_async_copy(gbuf_v.at[b],
                                  agg_sh.at[rowidx_v.at[j]],
                                  ssems[b]).wait()

        pltpu.sync_copy(col_hbm.at[pl.ds(s * EPT, EPT)], colidx_v)
        pltpu.sync_copy(row4_hbm.at[s], rowidx_v)
        _fire_g(0, 0)
        _fire_g(1, 1)

        def _scat_sync(j, b):
            pltpu.sync_copy(gbuf_v.at[b], agg_sh.at[rowidx_v.at[j]],
                            add=True)

        def _outer(g, carry):
            j0 = g * NBUF
            _wait_g(j0, 0)
            _scat_sync(j0, 0)
            _fire_g(j0 + NBUF, 0)
            j1 = j0 + 1
            _wait_g(j1, 1)
            _scat_sync(j1, 1)

            @pl.when(g < (NBLK - 1) // NBUF - 1)
            def _():
                _fire_g(j1 + NBUF, 1)

            return carry

        lax.fori_loop(0, (NBLK - 1) // NBUF, _outer, 0)
        _wait_g(NBLK - 1, 0)
        _scat_sync(NBLK - 1, 0)
        plsc.subcore_barrier()

        @pl.when(s < NS - 1)
        def _():
            _drain_rows(s * ROWS_A, CH_A, out_hbm)

        @pl.when(s == NS - 1)
        def _():
            _drain_rows(15 * ROWS_A, CH_B, out_hbm)

    @pl.when(c == 0)
    def _():
        _flow(z0_hbm, a0_hbm)

    @pl.when(c == 1)
    def _():
        _flow(z1_hbm, a1_hbm)


_spmm_call = functools.partial(
    pl.kernel,
    out_type=(jax.ShapeDtypeStruct((N, DH), jnp.float32),
              jax.ShapeDtypeStruct((N, DH), jnp.float32)),
    mesh=_mesh,
    scratch_types=(
        pltpu.VMEM_SHARED((N, DH), jnp.float32),
        pltpu.VMEM((EPT,), jnp.int32),
        pltpu.VMEM((NBLK, BLK), jnp.int32),
        pltpu.VMEM((NBUF, BLK, DH), jnp.float32),
        pltpu.SemaphoreType.DMA,
        pltpu.SemaphoreType.DMA,
        pltpu.SemaphoreType.DMA,
        pltpu.SemaphoreType.DMA,
    ),
)(_spmm_body)


def _out_body(a0_ref, a1_ref, w_ref, cdeg_ref, b_ref, o_ref):
    a = jnp.concatenate([a0_ref[...], a1_ref[...]], axis=1)
    y = lax.dot_general(a, w_ref[...], (((1,), (0,)), ((), ())),
                        preferred_element_type=jnp.float32,
                        precision=lax.Precision.HIGHEST)
    nr = lax.rsqrt(jnp.maximum(cdeg_ref[...], 1.0))
    o_ref[...] = y * nr + b_ref[...]


def _out_call(a0, a1, W, cdeg2d, bias2d):
    br = 2000
    return pl.pallas_call(
        _out_body,
        grid=(N // br,),
        in_specs=[pl.BlockSpec((br, DH), lambda i: (i, 0)),
                  pl.BlockSpec((br, DH), lambda i: (i, 0)),
                  pl.BlockSpec((D, D), lambda i: (0, 0)),
                  pl.BlockSpec((br, 1), lambda i: (i, 0)),
                  pl.BlockSpec((1, D), lambda i: (0, 0))],
        out_specs=pl.BlockSpec((br, D), lambda i: (i, 0)),
        out_shape=jax.ShapeDtypeStruct((N, D), jnp.float32),
    )(a0, a1, W, cdeg2d, bias2d)


def kernel(feat, edge_index, W, bias):
    row = edge_index[0].astype(jnp.int32)
    col = edge_index[1].astype(jnp.int32)
    row3 = row.reshape(NS, NBLK, BLK)
    col3 = col.reshape(NS, NBLK, BLK)
    z2d = jnp.zeros((BLK, DH), jnp.float32)

    rdeg, cdeg = _deg_call(row3, col3)
    z0, z1 = _norm_call(feat, rdeg.reshape(N, 1))
    a0, a1 = _spmm_call(z0, z1, col, row3, z2d)
    return _out_call(a0, a1, W, cdeg.reshape(N, 1), bias.reshape(1, D))
